# Initial kernel scaffold; baseline (speedup 1.0000x reference)
import jax
import jax.numpy as jnp
from jax.experimental import pallas as pl


def kernel(x, edge_index, edge_weight, W1, b1, W2, b2):
    raise NotImplementedError("write your pallas kernel here")


# speedup vs baseline: 7.9498x; 7.9498x over previous
import functools

import jax
import jax.numpy as jnp
from jax import lax
from jax.experimental import pallas as pl
from jax.experimental.pallas import tpu as pltpu
from jax.experimental.pallas import tpu_sc as plsc

NN = 10000
EE = 320000
NP = 10240
NC = 2
NS = 16
EB = 80
TSL = NP // NS

_mesh = plsc.VectorSubcoreMesh(core_axis_name="c", subcore_axis_name="s",
                               num_cores=NC, num_subcores=NS)


def _zero_rows(buf, nrows, ncol16):
    z = jnp.zeros((16,), jnp.float32)

    def body(i, _):
        for j in range(ncol16):
            buf[i, pl.ds(j * 16, 16)] = z
        return 0

    lax.fori_loop(0, nrows, body, 0)


def _deg_body(dst_hbm, w_hbm, out_hbm, acc_sh, dstb, wb, zb):
    c = lax.axis_index("c")
    t = lax.axis_index("s")

    _zero_rows(zb, 1, TSL // 16)
    pltpu.sync_copy(zb.at[0], acc_sh.at[pl.ds(t * TSL, TSL)])
    plsc.subcore_barrier()

    epc = EE // NC
    ept = epc // NS
    base = c * epc + t * ept
    nchunk = ept // EB

    def chunk(k, _):
        e0 = base + k * EB
        pltpu.sync_copy(dst_hbm.at[pl.ds(e0, EB)], dstb.at[0])
        pltpu.sync_copy(w_hbm.at[pl.ds(e0, EB)], wb)
        pltpu.sync_copy(wb, acc_sh.at[dstb.at[0]], add=True)
        return 0

    lax.fori_loop(0, nchunk, chunk, 0)
    plsc.subcore_barrier()
    pltpu.sync_copy(acc_sh.at[pl.ds(t * TSL, TSL)],
                    out_hbm.at[pl.ds(c * NP + t * TSL, TSL)])


@functools.partial(
    pl.kernel,
    out_type=jax.ShapeDtypeStruct((NC * NP,), jnp.float32),
    mesh=_mesh,
    scratch_types=[
        pltpu.VMEM_SHARED((NP,), jnp.float32),
        pltpu.VMEM((1, EB), jnp.int32),
        pltpu.VMEM((EB,), jnp.float32),
        pltpu.VMEM((1, TSL), jnp.float32),
    ],
)
def _deg_kernel(dst_hbm, w_hbm, out_hbm, acc_sh, dstb, wb, zb):
    _deg_body(dst_hbm, w_hbm, out_hbm, acc_sh, dstb, wb, zb)


def _agg_body(feat_split, src_hbm, dst_hbm, w_hbm, tab_hbm, out_hbm,
              acc_sh, srcb, adjb, dstb, wb, rows, zrow, gsem):
    c = lax.axis_index("c")
    t = lax.axis_index("s")

    _zero_rows(zrow, EB, 8)
    for j in range(TSL // EB):
        pltpu.sync_copy(zrow, acc_sh.at[pl.ds(t * TSL + j * EB, EB)])
    plsc.subcore_barrier()

    if feat_split:
        ept = EE // NS
        base = t * ept
        off = c * NN
    else:
        epc = EE // NC
        ept = epc // NS
        base = c * epc + t * ept
        off = 0
    nchunk = ept // EB

    def scale(g, _):
        w16 = wb[pl.ds(g * 16, 16)]
        for i in range(16):
            e = g * 16 + i
            s = jnp.full((16,), w16[i], jnp.float32)
            for j in range(8):
                rows[e, pl.ds(j * 16, 16)] = rows[e, pl.ds(j * 16, 16)] * s
        return 0

    def chunk(k, _):
        e0 = base + k * EB
        pltpu.sync_copy(src_hbm.at[pl.ds(e0, EB)], srcb)
        pltpu.sync_copy(dst_hbm.at[pl.ds(e0, EB)], dstb.at[0])
        pltpu.sync_copy(w_hbm.at[pl.ds(e0, EB)], wb)
        if feat_split:
            for g in range(EB // 16):
                adjb[pl.ds(g * 16, 16)] = srcb[pl.ds(g * 16, 16)] + off
            pltpu.async_copy(tab_hbm.at[adjb], rows, gsem).wait()
        else:
            pltpu.async_copy(tab_hbm.at[srcb], rows, gsem).wait()
        lax.fori_loop(0, EB // 16, scale, 0)
        pltpu.sync_copy(rows, acc_sh.at[dstb.at[0]], add=True)
        return 0

    lax.fori_loop(0, nchunk, chunk, 0)
    plsc.subcore_barrier()
    pltpu.sync_copy(acc_sh.at[pl.ds(t * TSL, TSL)],
                    out_hbm.at[pl.ds(c * NP + t * TSL, TSL)])


def _make_agg(feat_split, tab_rows):
    @functools.partial(
        pl.kernel,
        out_type=jax.ShapeDtypeStruct((NC * NP, 128), jnp.float32),
        mesh=_mesh,
        scratch_types=[
            pltpu.VMEM_SHARED((NP, 128), jnp.float32),
            pltpu.VMEM((EB,), jnp.int32),
            pltpu.VMEM((EB,), jnp.int32),
            pltpu.VMEM((1, EB), jnp.int32),
            pltpu.VMEM((EB,), jnp.float32),
            pltpu.VMEM((EB, 128), jnp.float32),
            pltpu.VMEM((EB, 128), jnp.float32),
            pltpu.SemaphoreType.DMA,
        ],
    )
    def agg(src_hbm, dst_hbm, w_hbm, tab_hbm, out_hbm, *scratch):
        _agg_body(feat_split, src_hbm, dst_hbm, w_hbm, tab_hbm, out_hbm,
                  *scratch)

    return agg


_agg_feat = _make_agg(True, 2 * NN)
_agg_edge = _make_agg(False, NN)


_RB = 512


def _dinv(dega_ref, degb_ref):
    return lax.rsqrt(1.0 + dega_ref[...] + degb_ref[...])[:, None]


def _tc1_body(x_ref, w1_ref, dega_ref, degb_ref, out_ref):
    dinv = _dinv(dega_ref, degb_ref)
    h = jnp.dot(x_ref[...], w1_ref[...], preferred_element_type=jnp.float32)
    out_ref[0] = h * dinv


def _tc2_body(agg_ref, h1s_ref, dega_ref, degb_ref, w2_ref, b1_ref, out_ref):
    dinv = _dinv(dega_ref, degb_ref)
    a0 = jnp.maximum((agg_ref[0] + h1s_ref[0]) * dinv + b1_ref[pl.ds(0, 128)][None, :], 0.0)
    a1 = jnp.maximum((agg_ref[1] + h1s_ref[1]) * dinv + b1_ref[pl.ds(128, 128)][None, :], 0.0)
    h2 = (jnp.dot(a0, w2_ref[pl.ds(0, 128), :], preferred_element_type=jnp.float32)
          + jnp.dot(a1, w2_ref[pl.ds(128, 128), :], preferred_element_type=jnp.float32))
    out_ref[...] = h2 * dinv


def _tc3_body(agg_ref, h2s_ref, dega_ref, degb_ref, b2_ref, out_ref):
    dinv = _dinv(dega_ref, degb_ref)
    out_ref[...] = (agg_ref[0] + agg_ref[1] + h2s_ref[...]) * dinv + b2_ref[...][None, :]


def kernel(x, edge_index, edge_weight, W1, b1, W2, b2):
    src = edge_index[0]
    dst = edge_index[1]

    degp = _deg_kernel(dst, edge_weight)

    h1s = pl.pallas_call(
        _tc1_body,
        grid=(2, NP // _RB),
        in_specs=[
            pl.BlockSpec((_RB, 128), lambda h, i: (i, 0)),
            pl.BlockSpec((128, 128), lambda h, i: (0, h)),
            pl.BlockSpec((_RB,), lambda h, i: (i,)),
            pl.BlockSpec((_RB,), lambda h, i: (i + NP // _RB,)),
        ],
        out_specs=pl.BlockSpec((1, _RB, 128), lambda h, i: (h, i, 0)),
        out_shape=jax.ShapeDtypeStruct((2, NN, 128), jnp.float32),
    )(x, W1, degp, degp)
    h1s_tab = h1s.reshape(2 * NN, 128)

    agg1 = _agg_feat(src, dst, edge_weight, h1s_tab)
    agg1 = agg1.reshape(2, NP, 128)

    h2s = pl.pallas_call(
        _tc2_body,
        grid=(NP // _RB,),
        in_specs=[
            pl.BlockSpec((2, _RB, 128), lambda i: (0, i, 0)),
            pl.BlockSpec((2, _RB, 128), lambda i: (0, i, 0)),
            pl.BlockSpec((_RB,), lambda i: (i,)),
            pl.BlockSpec((_RB,), lambda i: (i + NP // _RB,)),
            pl.BlockSpec((256, 128), lambda i: (0, 0)),
            pl.BlockSpec((256,), lambda i: (0,)),
        ],
        out_specs=pl.BlockSpec((_RB, 128), lambda i: (i, 0)),
        out_shape=jax.ShapeDtypeStruct((NN, 128), jnp.float32),
    )(agg1, h1s, degp, degp, W2, b1)

    agg2 = _agg_edge(src, dst, edge_weight, h2s)
    agg2 = agg2.reshape(2, NP, 128)

    out = pl.pallas_call(
        _tc3_body,
        grid=(NP // _RB,),
        in_specs=[
            pl.BlockSpec((2, _RB, 128), lambda i: (0, i, 0)),
            pl.BlockSpec((_RB, 128), lambda i: (i, 0)),
            pl.BlockSpec((_RB,), lambda i: (i,)),
            pl.BlockSpec((_RB,), lambda i: (i + NP // _RB,)),
            pl.BlockSpec((128,), lambda i: (0,)),
        ],
        out_specs=pl.BlockSpec((_RB, 128), lambda i: (i, 0)),
        out_shape=jax.ShapeDtypeStruct((NN, 128), jnp.float32),
    )(agg2, h2s, degp, degp, b2)
    return out


# speedup vs baseline: 19.7395x; 2.4830x over previous
import functools

import jax
import jax.numpy as jnp
from jax import lax
from jax.experimental import pallas as pl
from jax.experimental.pallas import tpu as pltpu
from jax.experimental.pallas import tpu_sc as plsc

NN = 10000
EE = 320000
PE = 327680
NP = 10240
NC = 2
NS = 16
EB = 128
ROWS = PE // EB
TSL = NP // NS

_mesh = plsc.VectorSubcoreMesh(core_axis_name="c", subcore_axis_name="s",
                               num_cores=NC, num_subcores=NS)


def _zero_chunk(buf):
    z = jnp.zeros((16,), jnp.float32)

    def body(i, _):
        for j in range(8):
            buf[i, pl.ds(j * 16, 16)] = z
        return 0

    lax.fori_loop(0, EB, body, 0)


def _deg_body(dst_hbm, w_hbm, out_hbm, acc_sh, dstb, wb, zb):
    c = lax.axis_index("c")
    t = lax.axis_index("s")

    def zbody(i, _):
        zb[0, pl.ds(i * 16, 16)] = jnp.zeros((16,), jnp.float32)
        return 0

    lax.fori_loop(0, TSL // 16, zbody, 0)
    pltpu.sync_copy(zb.at[0], acc_sh.at[pl.ds(t * TSL, TSL)])
    plsc.subcore_barrier()

    rpt = ROWS // (NC * NS)
    base = (c * NS + t) * rpt

    pltpu.sync_copy(dst_hbm.at[pl.ds(base, rpt)], dstb)
    pltpu.sync_copy(w_hbm.at[pl.ds(base, rpt)], wb)

    def chunk(k, _):
        pltpu.sync_copy(wb.at[k], acc_sh.at[dstb.at[k]], add=True)
        return 0

    lax.fori_loop(0, rpt, chunk, 0)
    plsc.subcore_barrier()
    pltpu.sync_copy(acc_sh.at[pl.ds(t * TSL, TSL)],
                    out_hbm.at[pl.ds(c * NP + t * TSL, TSL)])


@functools.partial(
    pl.kernel,
    out_type=jax.ShapeDtypeStruct((NC * NP,), jnp.float32),
    mesh=_mesh,
    scratch_types=[
        pltpu.VMEM_SHARED((NP,), jnp.float32),
        pltpu.VMEM((ROWS // (NC * NS), EB), jnp.int32),
        pltpu.VMEM((ROWS // (NC * NS), EB), jnp.float32),
        pltpu.VMEM((1, TSL), jnp.float32),
    ],
)
def _deg_kernel(dst_hbm, w_hbm, out_hbm, acc_sh, dstb, wb, zb):
    _deg_body(dst_hbm, w_hbm, out_hbm, acc_sh, dstb, wb, zb)


def _agg_body(feat_split, src_hbm, dst_hbm, w_hbm, tab_hbm, out_hbm,
              acc_sh, s0, s1, d0, d1, w0, w1, r0, r1,
              isem0, isem1, gsem0, gsem1):
    c = lax.axis_index("c")
    t = lax.axis_index("s")

    _zero_chunk(r0)
    for j in range(TSL // EB):
        pltpu.sync_copy(r0, acc_sh.at[pl.ds(t * TSL + j * EB, EB)])
    plsc.subcore_barrier()

    if feat_split:
        rpt = ROWS // NS
        rbase = c * ROWS + t * rpt
        dwbase = t * rpt
    else:
        rpt = ROWS // (NC * NS)
        rbase = (c * NS + t) * rpt
        dwbase = rbase

    srcs = (s0, s1)
    dsts = (d0, d1)
    ws = (w0, w1)
    rows = (r0, r1)
    isems = (isem0, isem1)
    gsems = (gsem0, gsem1)

    def in_start(u, b):
        pltpu.async_copy(src_hbm.at[pl.ds(rbase + u, 1)], srcs[b], isems[b])
        pltpu.async_copy(dst_hbm.at[pl.ds(dwbase + u, 1)], dsts[b], isems[b])
        pltpu.async_copy(w_hbm.at[pl.ds(dwbase + u, 1)], ws[b], isems[b])

    def in_wait(b):
        pltpu.make_async_copy(src_hbm.at[pl.ds(0, 1)], srcs[b], isems[b]).wait()
        pltpu.make_async_copy(dst_hbm.at[pl.ds(0, 1)], dsts[b], isems[b]).wait()
        pltpu.make_async_copy(w_hbm.at[pl.ds(0, 1)], ws[b], isems[b]).wait()

    def g_start(b):
        pltpu.async_copy(tab_hbm.at[srcs[b].at[0]], rows[b], gsems[b])

    def g_wait(b):
        pltpu.make_async_copy(tab_hbm.at[srcs[b].at[0]], rows[b],
                              gsems[b]).wait()

    def make_scale(b):
        rb = rows[b]
        wbuf = ws[b]

        def scale(g, _):
            w16 = wbuf[0, pl.ds(g * 16, 16)]
            for i in range(16):
                s = jnp.full((16,), w16[i], jnp.float32)
                for cc in range(8):
                    rb[g * 16 + i, pl.ds(cc * 16, 16)] = (
                        rb[g * 16 + i, pl.ds(cc * 16, 16)] * s)
            return 0

        return scale

    scales = (make_scale(0), make_scale(1))

    in_start(0, 0)
    in_start(1, 1)
    in_wait(0)
    g_start(0)

    def iteration(s, _):
        for b in range(2):
            u = 2 * s + b
            g_wait(b)
            @pl.when(u + 1 < rpt)
            def _():
                in_wait(1 - b)
                g_start(1 - b)
            lax.fori_loop(0, EB // 16, scales[b], 0)
            pltpu.sync_copy(rows[b], acc_sh.at[dsts[b].at[0]], add=True)
            @pl.when(u + 2 < rpt)
            def _():
                in_start(u + 2, b)
        return 0

    lax.fori_loop(0, rpt // 2, iteration, 0)
    plsc.subcore_barrier()
    pltpu.sync_copy(acc_sh.at[pl.ds(t * TSL, TSL)],
                    out_hbm.at[pl.ds(c * NP + t * TSL, TSL)])


def _make_agg(feat_split):
    @functools.partial(
        pl.kernel,
        out_type=jax.ShapeDtypeStruct((NC * NP, 128), jnp.float32),
        mesh=_mesh,
        scratch_types=[
            pltpu.VMEM_SHARED((NP, 128), jnp.float32),
            pltpu.VMEM((1, EB), jnp.int32),
            pltpu.VMEM((1, EB), jnp.int32),
            pltpu.VMEM((1, EB), jnp.int32),
            pltpu.VMEM((1, EB), jnp.int32),
            pltpu.VMEM((1, EB), jnp.float32),
            pltpu.VMEM((1, EB), jnp.float32),
            pltpu.VMEM((EB, 128), jnp.float32),
            pltpu.VMEM((EB, 128), jnp.float32),
            pltpu.SemaphoreType.DMA,
            pltpu.SemaphoreType.DMA,
            pltpu.SemaphoreType.DMA,
            pltpu.SemaphoreType.DMA,
        ],
    )
    def agg(src_hbm, dst_hbm, w_hbm, tab_hbm, out_hbm, *scratch):
        _agg_body(feat_split, src_hbm, dst_hbm, w_hbm, tab_hbm, out_hbm,
                  *scratch)

    return agg


_agg_feat = _make_agg(True)
_agg_edge = _make_agg(False)


_RB = 512


def _dinv(dega_ref, degb_ref):
    return lax.rsqrt(1.0 + dega_ref[...] + degb_ref[...])[:, None]


def _tc1_body(x_ref, w1_ref, dega_ref, degb_ref, out_ref):
    dinv = _dinv(dega_ref, degb_ref)
    h = jnp.dot(x_ref[...], w1_ref[...], preferred_element_type=jnp.float32)
    out_ref[0] = h * dinv


def _tc2_body(agg_ref, h1s_ref, dega_ref, degb_ref, w2_ref, b1_ref, out_ref):
    dinv = _dinv(dega_ref, degb_ref)
    a0 = jnp.maximum((agg_ref[0] + h1s_ref[0]) * dinv + b1_ref[pl.ds(0, 128)][None, :], 0.0)
    a1 = jnp.maximum((agg_ref[1] + h1s_ref[1]) * dinv + b1_ref[pl.ds(128, 128)][None, :], 0.0)
    h2 = (jnp.dot(a0, w2_ref[pl.ds(0, 128), :], preferred_element_type=jnp.float32)
          + jnp.dot(a1, w2_ref[pl.ds(128, 128), :], preferred_element_type=jnp.float32))
    out_ref[...] = h2 * dinv


def _tc3_body(agg_ref, h2s_ref, dega_ref, degb_ref, b2_ref, out_ref):
    dinv = _dinv(dega_ref, degb_ref)
    out_ref[...] = (agg_ref[0] + agg_ref[1] + h2s_ref[...]) * dinv + b2_ref[...][None, :]


def kernel(x, edge_index, edge_weight, W1, b1, W2, b2):
    pad = PE - EE
    padidx = (jnp.arange(pad, dtype=jnp.int32) * 13) % NN
    src_p = jnp.concatenate([edge_index[0], padidx])
    dst_p = jnp.concatenate([edge_index[1], padidx])
    w_p = jnp.concatenate([edge_weight, jnp.zeros((pad,), jnp.float32)])
    src_tab = jnp.concatenate([src_p, src_p + NN]).reshape(2 * ROWS, EB)
    src2d = src_p.reshape(ROWS, EB)
    dst2d = dst_p.reshape(ROWS, EB)
    w2d = w_p.reshape(ROWS, EB)

    degp = _deg_kernel(dst2d, w2d)

    h1s = pl.pallas_call(
        _tc1_body,
        grid=(2, NP // _RB),
        in_specs=[
            pl.BlockSpec((_RB, 128), lambda h, i: (i, 0)),
            pl.BlockSpec((128, 128), lambda h, i: (0, h)),
            pl.BlockSpec((_RB,), lambda h, i: (i,)),
            pl.BlockSpec((_RB,), lambda h, i: (i + NP // _RB,)),
        ],
        out_specs=pl.BlockSpec((1, _RB, 128), lambda h, i: (h, i, 0)),
        out_shape=jax.ShapeDtypeStruct((2, NN, 128), jnp.float32),
    )(x, W1, degp, degp)
    h1s_tab = h1s.reshape(2 * NN, 128)

    agg1 = _agg_feat(src_tab, dst2d, w2d, h1s_tab)
    agg1 = agg1.reshape(2, NP, 128)

    h2s = pl.pallas_call(
        _tc2_body,
        grid=(NP // _RB,),
        in_specs=[
            pl.BlockSpec((2, _RB, 128), lambda i: (0, i, 0)),
            pl.BlockSpec((2, _RB, 128), lambda i: (0, i, 0)),
            pl.BlockSpec((_RB,), lambda i: (i,)),
            pl.BlockSpec((_RB,), lambda i: (i + NP // _RB,)),
            pl.BlockSpec((256, 128), lambda i: (0, 0)),
            pl.BlockSpec((256,), lambda i: (0,)),
        ],
        out_specs=pl.BlockSpec((_RB, 128), lambda i: (i, 0)),
        out_shape=jax.ShapeDtypeStruct((NN, 128), jnp.float32),
    )(agg1, h1s, degp, degp, W2, b1)

    agg2 = _agg_edge(src2d, dst2d, w2d, h2s)
    agg2 = agg2.reshape(2, NP, 128)

    out = pl.pallas_call(
        _tc3_body,
        grid=(NP // _RB,),
        in_specs=[
            pl.BlockSpec((2, _RB, 128), lambda i: (0, i, 0)),
            pl.BlockSpec((_RB, 128), lambda i: (i, 0)),
            pl.BlockSpec((_RB,), lambda i: (i,)),
            pl.BlockSpec((_RB,), lambda i: (i + NP // _RB,)),
            pl.BlockSpec((128,), lambda i: (0,)),
        ],
        out_specs=pl.BlockSpec((_RB, 128), lambda i: (i, 0)),
        out_shape=jax.ShapeDtypeStruct((NN, 128), jnp.float32),
    )(agg2, h2s, degp, degp, b2)
    return out


# speedup vs baseline: 23.0025x; 1.1653x over previous
import functools

import jax
import jax.numpy as jnp
from jax import lax
from jax.experimental import pallas as pl
from jax.experimental.pallas import tpu as pltpu
from jax.experimental.pallas import tpu_sc as plsc

NN = 10000
EE = 320000
PE = 327680
NP = 10240
NC = 2
NS = 16
EB = 128
ROWS = PE // EB
TSL = NP // NS

_mesh = plsc.VectorSubcoreMesh(core_axis_name="c", subcore_axis_name="s",
                               num_cores=NC, num_subcores=NS)


def _zero_chunk(buf):
    z = jnp.zeros((16,), jnp.float32)

    def body(i, _):
        for j in range(8):
            buf[i, pl.ds(j * 16, 16)] = z
        return 0

    lax.fori_loop(0, EB, body, 0)


def _deg_body(dst_hbm, w_hbm, out_hbm, acc_sh, dstb, wb, zb):
    c = lax.axis_index("c")
    t = lax.axis_index("s")

    def zbody(i, _):
        zb[0, pl.ds(i * 16, 16)] = jnp.zeros((16,), jnp.float32)
        return 0

    lax.fori_loop(0, TSL // 16, zbody, 0)
    pltpu.sync_copy(zb.at[0], acc_sh.at[pl.ds(t * TSL, TSL)])
    plsc.subcore_barrier()

    rpt = ROWS // (NC * NS)
    base = (c * NS + t) * rpt

    pltpu.sync_copy(dst_hbm.at[pl.ds(base, rpt)], dstb)
    pltpu.sync_copy(w_hbm.at[pl.ds(base, rpt)], wb)

    def chunk(k, _):
        pltpu.sync_copy(wb.at[k], acc_sh.at[dstb.at[k]], add=True)
        return 0

    lax.fori_loop(0, rpt, chunk, 0)
    plsc.subcore_barrier()
    pltpu.sync_copy(acc_sh.at[pl.ds(t * TSL, TSL)],
                    out_hbm.at[pl.ds(c * NP + t * TSL, TSL)])


@functools.partial(
    pl.kernel,
    out_type=jax.ShapeDtypeStruct((NC * NP,), jnp.float32),
    mesh=_mesh,
    scratch_types=[
        pltpu.VMEM_SHARED((NP,), jnp.float32),
        pltpu.VMEM((ROWS // (NC * NS), EB), jnp.int32),
        pltpu.VMEM((ROWS // (NC * NS), EB), jnp.float32),
        pltpu.VMEM((1, TSL), jnp.float32),
    ],
)
def _deg_kernel(dst_hbm, w_hbm, out_hbm, acc_sh, dstb, wb, zb):
    _deg_body(dst_hbm, w_hbm, out_hbm, acc_sh, dstb, wb, zb)


def _agg_body(feat_split, src_hbm, dst_hbm, w_hbm, tab_hbm, out_hbm,
              acc_sh, s0, s1, d0, d1, w0, w1, r0, r1, sd0, sd1,
              isem0, isem1, gsem0, gsem1, ssem0, ssem1):
    c = lax.axis_index("c")
    t = lax.axis_index("s")

    _zero_chunk(r0)
    for j in range(TSL // EB):
        pltpu.sync_copy(r0, acc_sh.at[pl.ds(t * TSL + j * EB, EB)])
    plsc.subcore_barrier()

    if feat_split:
        rpt = ROWS // NS
        rbase = c * ROWS + t * rpt
        dwbase = t * rpt
    else:
        rpt = ROWS // (NC * NS)
        rbase = (c * NS + t) * rpt
        dwbase = rbase

    srcs = (s0, s1)
    dsts = (d0, d1)
    ws = (w0, w1)
    rows = (r0, r1)
    sds = (sd0, sd1)
    isems = (isem0, isem1)
    gsems = (gsem0, gsem1)
    ssems = (ssem0, ssem1)

    def in_start(u, b):
        pltpu.async_copy(src_hbm.at[pl.ds(rbase + u, 1)], srcs[b], isems[b])
        pltpu.async_copy(dst_hbm.at[pl.ds(dwbase + u, 1)], dsts[b], isems[b])
        pltpu.async_copy(w_hbm.at[pl.ds(dwbase + u, 1)], ws[b], isems[b])

    def in_wait(b):
        pltpu.make_async_copy(src_hbm.at[pl.ds(0, 1)], srcs[b], isems[b]).wait()
        pltpu.make_async_copy(dst_hbm.at[pl.ds(0, 1)], dsts[b], isems[b]).wait()
        pltpu.make_async_copy(w_hbm.at[pl.ds(0, 1)], ws[b], isems[b]).wait()

    def g_start(b):
        pltpu.async_copy(tab_hbm.at[srcs[b].at[0]], rows[b], gsems[b])

    def g_wait(b):
        pltpu.make_async_copy(tab_hbm.at[srcs[b].at[0]], rows[b],
                              gsems[b]).wait()

    def s_start(b):
        pltpu.async_copy(rows[b], acc_sh.at[sds[b].at[0]], ssems[b], add=True)

    def s_wait(b):
        pltpu.make_async_copy(rows[b], acc_sh.at[sds[b].at[0]],
                              ssems[b]).wait()

    def make_scale(b):
        rb = rows[b]
        wbuf = ws[b]

        def scale(g, _):
            w16 = wbuf[0, pl.ds(g * 16, 16)]
            for i in range(16):
                s = jnp.full((16,), w16[i], jnp.float32)
                for cc in range(8):
                    rb[g * 16 + i, pl.ds(cc * 16, 16)] = (
                        rb[g * 16 + i, pl.ds(cc * 16, 16)] * s)
            return 0

        return scale

    scales = (make_scale(0), make_scale(1))

    in_start(0, 0)
    in_start(1, 1)
    in_wait(0)
    g_start(0)

    def iteration(s, _):
        for b in range(2):
            u = 2 * s + b
            g_wait(b)
            @pl.when(u >= 1)
            def _():
                s_wait(1 - b)
            @pl.when(u + 1 < rpt)
            def _():
                in_wait(1 - b)
                g_start(1 - b)
            lax.fori_loop(0, EB // 16, scales[b], 0)
            for cc in range(8):
                sds[b][0, pl.ds(cc * 16, 16)] = dsts[b][0, pl.ds(cc * 16, 16)]
            s_start(b)
            @pl.when(u + 2 < rpt)
            def _():
                in_start(u + 2, b)
        return 0

    lax.fori_loop(0, rpt // 2, iteration, 0)
    s_wait((rpt - 1) % 2)
    plsc.subcore_barrier()
    pltpu.sync_copy(acc_sh.at[pl.ds(t * TSL, TSL)],
                    out_hbm.at[pl.ds(c * NP + t * TSL, TSL)])


def _make_agg(feat_split):
    @functools.partial(
        pl.kernel,
        out_type=jax.ShapeDtypeStruct((NC * NP, 128), jnp.float32),
        mesh=_mesh,
        scratch_types=[
            pltpu.VMEM_SHARED((NP, 128), jnp.float32),
            pltpu.VMEM((1, EB), jnp.int32),
            pltpu.VMEM((1, EB), jnp.int32),
            pltpu.VMEM((1, EB), jnp.int32),
            pltpu.VMEM((1, EB), jnp.int32),
            pltpu.VMEM((1, EB), jnp.float32),
            pltpu.VMEM((1, EB), jnp.float32),
            pltpu.VMEM((EB, 128), jnp.float32),
            pltpu.VMEM((EB, 128), jnp.float32),
            pltpu.VMEM((1, EB), jnp.int32),
            pltpu.VMEM((1, EB), jnp.int32),
            pltpu.SemaphoreType.DMA,
            pltpu.SemaphoreType.DMA,
            pltpu.SemaphoreType.DMA,
            pltpu.SemaphoreType.DMA,
            pltpu.SemaphoreType.DMA,
            pltpu.SemaphoreType.DMA,
        ],
    )
    def agg(src_hbm, dst_hbm, w_hbm, tab_hbm, out_hbm, *scratch):
        _agg_body(feat_split, src_hbm, dst_hbm, w_hbm, tab_hbm, out_hbm,
                  *scratch)

    return agg


_agg_feat = _make_agg(True)
_agg_edge = _make_agg(False)


_RB = 512


def _dinv(dega_ref, degb_ref):
    return lax.rsqrt(1.0 + dega_ref[...] + degb_ref[...])[:, None]


def _tc1_body(x_ref, w1_ref, dega_ref, degb_ref, out_ref):
    dinv = _dinv(dega_ref, degb_ref)
    h = jnp.dot(x_ref[...], w1_ref[...], preferred_element_type=jnp.float32)
    out_ref[0] = h * dinv


def _tc2_body(agg_ref, h1s_ref, dega_ref, degb_ref, w2_ref, b1_ref, out_ref):
    dinv = _dinv(dega_ref, degb_ref)
    a0 = jnp.maximum((agg_ref[0] + h1s_ref[0]) * dinv + b1_ref[pl.ds(0, 128)][None, :], 0.0)
    a1 = jnp.maximum((agg_ref[1] + h1s_ref[1]) * dinv + b1_ref[pl.ds(128, 128)][None, :], 0.0)
    h2 = (jnp.dot(a0, w2_ref[pl.ds(0, 128), :], preferred_element_type=jnp.float32)
          + jnp.dot(a1, w2_ref[pl.ds(128, 128), :], preferred_element_type=jnp.float32))
    out_ref[...] = h2 * dinv


def _tc3_body(agg_ref, h2s_ref, dega_ref, degb_ref, b2_ref, out_ref):
    dinv = _dinv(dega_ref, degb_ref)
    out_ref[...] = (agg_ref[0] + agg_ref[1] + h2s_ref[...]) * dinv + b2_ref[...][None, :]


def kernel(x, edge_index, edge_weight, W1, b1, W2, b2):
    pad = PE - EE
    padidx = (jnp.arange(pad, dtype=jnp.int32) * 13) % NN
    src_p = jnp.concatenate([edge_index[0], padidx])
    dst_p = jnp.concatenate([edge_index[1], padidx])
    w_p = jnp.concatenate([edge_weight, jnp.zeros((pad,), jnp.float32)])
    src_tab = jnp.concatenate([src_p, src_p + NN]).reshape(2 * ROWS, EB)
    src2d = src_p.reshape(ROWS, EB)
    dst2d = dst_p.reshape(ROWS, EB)
    w2d = w_p.reshape(ROWS, EB)

    degp = _deg_kernel(dst2d, w2d)

    h1s = pl.pallas_call(
        _tc1_body,
        grid=(2, NP // _RB),
        in_specs=[
            pl.BlockSpec((_RB, 128), lambda h, i: (i, 0)),
            pl.BlockSpec((128, 128), lambda h, i: (0, h)),
            pl.BlockSpec((_RB,), lambda h, i: (i,)),
            pl.BlockSpec((_RB,), lambda h, i: (i + NP // _RB,)),
        ],
        out_specs=pl.BlockSpec((1, _RB, 128), lambda h, i: (h, i, 0)),
        out_shape=jax.ShapeDtypeStruct((2, NN, 128), jnp.float32),
    )(x, W1, degp, degp)
    h1s_tab = h1s.reshape(2 * NN, 128)

    agg1 = _agg_feat(src_tab, dst2d, w2d, h1s_tab)
    agg1 = agg1.reshape(2, NP, 128)

    h2s = pl.pallas_call(
        _tc2_body,
        grid=(NP // _RB,),
        in_specs=[
            pl.BlockSpec((2, _RB, 128), lambda i: (0, i, 0)),
            pl.BlockSpec((2, _RB, 128), lambda i: (0, i, 0)),
            pl.BlockSpec((_RB,), lambda i: (i,)),
            pl.BlockSpec((_RB,), lambda i: (i + NP // _RB,)),
            pl.BlockSpec((256, 128), lambda i: (0, 0)),
            pl.BlockSpec((256,), lambda i: (0,)),
        ],
        out_specs=pl.BlockSpec((_RB, 128), lambda i: (i, 0)),
        out_shape=jax.ShapeDtypeStruct((NN, 128), jnp.float32),
    )(agg1, h1s, degp, degp, W2, b1)

    agg2 = _agg_edge(src2d, dst2d, w2d, h2s)
    agg2 = agg2.reshape(2, NP, 128)

    out = pl.pallas_call(
        _tc3_body,
        grid=(NP // _RB,),
        in_specs=[
            pl.BlockSpec((2, _RB, 128), lambda i: (0, i, 0)),
            pl.BlockSpec((_RB, 128), lambda i: (i, 0)),
            pl.BlockSpec((_RB,), lambda i: (i,)),
            pl.BlockSpec((_RB,), lambda i: (i + NP // _RB,)),
            pl.BlockSpec((128,), lambda i: (0,)),
        ],
        out_specs=pl.BlockSpec((_RB, 128), lambda i: (i, 0)),
        out_shape=jax.ShapeDtypeStruct((NN, 128), jnp.float32),
    )(agg2, h2s, degp, degp, b2)
    return out
